# baseline (speedup 1.0000x reference)
import functools

import jax
import jax.numpy as jnp
from jax import lax
from jax.experimental import pallas as pl
from jax.experimental.pallas import tpu as pltpu
from jax.experimental.pallas import tpu_sc as plsc

N_NODES = 10000
D_FEAT = 128
N_EDGES = 320000

NC = 2
NS = 16
NW = NC * NS
CHUNK = 48
NBUF = 6
N_ROUNDS = 36
E_PER_W = N_ROUNDS * NBUF * CHUNK
E_PAD = NW * E_PER_W
N_PAD = 10496
ROWS_PER_TILE = N_PAD // NS


def _sc_partial_sums(x, src, dst, zrows):
    mesh = plsc.VectorSubcoreMesh(core_axis_name="c", subcore_axis_name="s")

    @functools.partial(
        pl.kernel,
        mesh=mesh,
        out_type=jax.ShapeDtypeStruct((NC, N_PAD, D_FEAT), jnp.float32),
        scratch_types=[
            pltpu.VMEM((3, NBUF, CHUNK), jnp.int32),
            pltpu.VMEM((3, NBUF, CHUNK), jnp.int32),
            pltpu.VMEM((NBUF, CHUNK, D_FEAT), jnp.float32),
            pltpu.VMEM_SHARED((N_PAD, D_FEAT), jnp.float32),
            pltpu.SemaphoreType.DMA((NBUF,)),
            pltpu.SemaphoreType.DMA((NBUF,)),
            pltpu.SemaphoreType.DMA,
            pltpu.SemaphoreType.DMA,
        ],
    )
    def k(x_hbm, src_hbm, dst_hbm, z_hbm, out_hbm, sidx, didx, rows, acc,
          gsem, ssem, isem_s, isem_d):
        cid = lax.axis_index("c")
        sid = lax.axis_index("s")
        wid = sid * NC + cid
        rbase = sid * ROWS_PER_TILE
        pltpu.sync_copy(src_hbm.at[wid, 0], sidx.at[0])
        pltpu.sync_copy(dst_hbm.at[wid, 0], didx.at[0])
        for b in range(NBUF):
            pltpu.async_copy(x_hbm.at[sidx.at[0, b]], rows.at[b], gsem.at[b])
        pltpu.async_copy(src_hbm.at[wid, 1], sidx.at[1], isem_s)
        pltpu.async_copy(dst_hbm.at[wid, 1], didx.at[1], isem_d)
        pltpu.sync_copy(z_hbm, acc.at[pl.ds(rbase, ROWS_PER_TILE)])
        plsc.subcore_barrier()

        def round_body(t, carry):
            p0 = lax.rem(t, 3)
            p1 = lax.rem(t + 1, 3)
            p2 = lax.rem(t + 2, 3)
            rr = jnp.minimum(t + 2, N_ROUNDS - 1)
            pltpu.async_copy(src_hbm.at[wid, rr], sidx.at[p2], isem_s)
            pltpu.async_copy(dst_hbm.at[wid, rr], didx.at[p2], isem_d)
            for b in range(NBUF):
                pltpu.make_async_copy(
                    x_hbm.at[sidx.at[p0, b]], rows.at[b], gsem.at[b]).wait()
                pltpu.async_copy(
                    rows.at[b], acc.at[didx.at[p0, b]], ssem.at[b], add=True)
            pltpu.make_async_copy(
                src_hbm.at[wid, rr], sidx.at[p1], isem_s).wait()
            pltpu.make_async_copy(
                dst_hbm.at[wid, rr], didx.at[p1], isem_d).wait()
            for b in range(NBUF):
                pltpu.make_async_copy(
                    rows.at[b], acc.at[didx.at[p0, b]], ssem.at[b]).wait()
                pltpu.async_copy(x_hbm.at[sidx.at[p1, b]], rows.at[b],
                                 gsem.at[b])
            return carry

        lax.fori_loop(0, N_ROUNDS, round_body, 0)
        for b in range(NBUF):
            pltpu.make_async_copy(
                x_hbm.at[sidx.at[0, b]], rows.at[b], gsem.at[b]).wait()
        pltpu.make_async_copy(src_hbm.at[wid, 0], sidx.at[2], isem_s).wait()
        pltpu.make_async_copy(dst_hbm.at[wid, 0], didx.at[2], isem_d).wait()
        plsc.subcore_barrier()
        pltpu.sync_copy(acc.at[pl.ds(rbase, ROWS_PER_TILE)],
                        out_hbm.at[cid, pl.ds(rbase, ROWS_PER_TILE)])

    return k(x, src, dst, zrows)


def _combine(partial):
    def body(p_ref, o_ref):
        o_ref[...] = p_ref[0] + p_ref[1]

    bs = 1000
    return pl.pallas_call(
        body,
        grid=(N_NODES // bs,),
        in_specs=[pl.BlockSpec((NC, bs, D_FEAT), lambda i: (0, i, 0))],
        out_specs=pl.BlockSpec((bs, D_FEAT), lambda i: (i, 0)),
        out_shape=jax.ShapeDtypeStruct((N_NODES, D_FEAT), jnp.float32),
    )(partial)


def kernel(x, edge_index):
    ei = edge_index.astype(jnp.int32)
    npad = E_PAD - N_EDGES
    npad_w = npad // NW
    dsrc = (jnp.arange(NW * npad_w, dtype=jnp.int32) % N_NODES
            ).reshape(NW, npad_w)
    ddst = N_NODES + (jnp.arange(NW * npad_w, dtype=jnp.int32)
                      % (N_PAD - N_NODES)).reshape(NW, npad_w)
    src = jnp.concatenate([ei[0].reshape(NW, -1), dsrc], axis=1)
    dst = jnp.concatenate([ei[1].reshape(NW, -1), ddst], axis=1)
    src = src.reshape(NW, N_ROUNDS, NBUF, CHUNK)
    dst = dst.reshape(NW, N_ROUNDS, NBUF, CHUNK)
    zrows = jnp.zeros((ROWS_PER_TILE, D_FEAT), jnp.float32)
    partial = _sc_partial_sums(x, src, dst, zrows)
    return _combine(partial)


# speedup vs baseline: 1.0016x; 1.0016x over previous
import functools

import jax
import jax.numpy as jnp
from jax import lax
from jax.experimental import pallas as pl
from jax.experimental.pallas import tpu as pltpu
from jax.experimental.pallas import tpu_sc as plsc

N_NODES = 10000
D_FEAT = 128
N_EDGES = 320000

NC = 2
NS = 16
NW = NC * NS
CHUNK = 48
NBUF = 5
N_ROUNDS = 42
E_PER_W = N_ROUNDS * NBUF * CHUNK
E_PAD = NW * E_PER_W
N_PAD = 11264
ROWS_PER_TILE = N_PAD // NS


def _sc_partial_sums(x, src, dst, zrows):
    mesh = plsc.VectorSubcoreMesh(core_axis_name="c", subcore_axis_name="s")

    @functools.partial(
        pl.kernel,
        mesh=mesh,
        out_type=jax.ShapeDtypeStruct((NC, N_PAD, D_FEAT), jnp.float32),
        scratch_types=[
            pltpu.VMEM((3, NBUF, CHUNK), jnp.int32),
            pltpu.VMEM((3, NBUF, CHUNK), jnp.int32),
            pltpu.VMEM((NBUF, CHUNK, D_FEAT), jnp.float32),
            pltpu.VMEM_SHARED((N_PAD, D_FEAT), jnp.float32),
            pltpu.SemaphoreType.DMA((NBUF,)),
            pltpu.SemaphoreType.DMA((NBUF,)),
            pltpu.SemaphoreType.DMA,
            pltpu.SemaphoreType.DMA,
        ],
    )
    def k(x_hbm, src_hbm, dst_hbm, z_hbm, out_hbm, sidx, didx, rows, acc,
          gsem, ssem, isem_s, isem_d):
        cid = lax.axis_index("c")
        sid = lax.axis_index("s")
        wid = sid * NC + cid
        rbase = sid * ROWS_PER_TILE
        pltpu.sync_copy(src_hbm.at[wid, 0], sidx.at[0])
        pltpu.sync_copy(dst_hbm.at[wid, 0], didx.at[0])
        for b in range(NBUF):
            pltpu.async_copy(x_hbm.at[sidx.at[0, b]], rows.at[b], gsem.at[b])
        pltpu.async_copy(src_hbm.at[wid, 1], sidx.at[1], isem_s)
        pltpu.async_copy(dst_hbm.at[wid, 1], didx.at[1], isem_d)
        pltpu.sync_copy(z_hbm, acc.at[pl.ds(rbase, ROWS_PER_TILE)])
        plsc.subcore_barrier()

        def round_body(t, carry):
            p0 = lax.rem(t, 3)
            p1 = lax.rem(t + 1, 3)
            p2 = lax.rem(t + 2, 3)
            rr = jnp.minimum(t + 2, N_ROUNDS - 1)
            pltpu.async_copy(src_hbm.at[wid, rr], sidx.at[p2], isem_s)
            pltpu.async_copy(dst_hbm.at[wid, rr], didx.at[p2], isem_d)
            for b in range(NBUF):
                pltpu.make_async_copy(
                    x_hbm.at[sidx.at[p0, b]], rows.at[b], gsem.at[b]).wait()
                pltpu.async_copy(
                    rows.at[b], acc.at[didx.at[p0, b]], ssem.at[b], add=True)
            pltpu.make_async_copy(
                src_hbm.at[wid, rr], sidx.at[p1], isem_s).wait()
            pltpu.make_async_copy(
                dst_hbm.at[wid, rr], didx.at[p1], isem_d).wait()
            for b in range(NBUF):
                pltpu.make_async_copy(
                    rows.at[b], acc.at[didx.at[p0, b]], ssem.at[b]).wait()
                pltpu.async_copy(x_hbm.at[sidx.at[p1, b]], rows.at[b],
                                 gsem.at[b])
            return carry

        lax.fori_loop(0, N_ROUNDS, round_body, 0)
        for b in range(NBUF):
            pltpu.make_async_copy(
                x_hbm.at[sidx.at[0, b]], rows.at[b], gsem.at[b]).wait()
        pltpu.make_async_copy(src_hbm.at[wid, 0], sidx.at[2], isem_s).wait()
        pltpu.make_async_copy(dst_hbm.at[wid, 0], didx.at[2], isem_d).wait()
        plsc.subcore_barrier()
        pltpu.sync_copy(acc.at[pl.ds(rbase, ROWS_PER_TILE)],
                        out_hbm.at[cid, pl.ds(rbase, ROWS_PER_TILE)])

    return k(x, src, dst, zrows)


def _combine(partial):
    def body(p_ref, o_ref):
        o_ref[...] = p_ref[0] + p_ref[1]

    bs = 1000
    return pl.pallas_call(
        body,
        grid=(N_NODES // bs,),
        in_specs=[pl.BlockSpec((NC, bs, D_FEAT), lambda i: (0, i, 0))],
        out_specs=pl.BlockSpec((bs, D_FEAT), lambda i: (i, 0)),
        out_shape=jax.ShapeDtypeStruct((N_NODES, D_FEAT), jnp.float32),
    )(partial)


def kernel(x, edge_index):
    ei = edge_index.astype(jnp.int32)
    npad = E_PAD - N_EDGES
    npad_w = npad // NW
    dsrc = (jnp.arange(NW * npad_w, dtype=jnp.int32) % N_NODES
            ).reshape(NW, npad_w)
    ddst = N_NODES + (jnp.arange(NW * npad_w, dtype=jnp.int32)
                      % (N_PAD - N_NODES)).reshape(NW, npad_w)
    src = jnp.concatenate([ei[0].reshape(NW, -1), dsrc], axis=1)
    dst = jnp.concatenate([ei[1].reshape(NW, -1), ddst], axis=1)
    src = src.reshape(NW, N_ROUNDS, NBUF, CHUNK)
    dst = dst.reshape(NW, N_ROUNDS, NBUF, CHUNK)
    zrows = jnp.zeros((ROWS_PER_TILE, D_FEAT), jnp.float32)
    partial = _sc_partial_sums(x, src, dst, zrows)
    return _combine(partial)
